# Initial kernel scaffold; baseline (speedup 1.0000x reference)
import jax
import jax.numpy as jnp
from jax.experimental import pallas as pl


def kernel(words_embeddings, token_type_ids, position_table, token_type_table, ln_weight, ln_bias):
    raise NotImplementedError("write your pallas kernel here")


# speedup vs baseline: 3.1880x; 3.1880x over previous
import jax
import jax.numpy as jnp
from jax.experimental import pallas as pl

B, S, H = 4, 8192, 1024
EPS = 1e-12
TILE_S = 256
NB = S // TILE_S


def _body(w_ref, pos_ref, tt_ref, ttab_ref, g_ref, b_ref, o_ref):
    x = w_ref[0] + pos_ref[...]
    row0 = ttab_ref[0:1, :]
    delta = ttab_ref[1:2, :] - row0
    ttf = tt_ref[0, 0].astype(jnp.float32)
    x = x + row0 + ttf * delta
    u = jnp.mean(x, axis=-1, keepdims=True)
    xc = x - u
    var = jnp.mean(xc * xc, axis=-1, keepdims=True)
    y = xc * jax.lax.rsqrt(var + EPS)
    o_ref[0] = y * g_ref[...] + b_ref[...]


def kernel(words_embeddings, token_type_ids, position_table, token_type_table, ln_weight, ln_bias):
    tt_t = token_type_ids.astype(jnp.int32).reshape(B, NB, TILE_S, 1)
    g = ln_weight.reshape(1, H)
    b = ln_bias.reshape(1, H)
    out = pl.pallas_call(
        _body,
        grid=(NB, B),
        in_specs=[
            pl.BlockSpec((1, TILE_S, H), lambda i, j: (j, i, 0)),
            pl.BlockSpec((TILE_S, H), lambda i, j: (i, 0)),
            pl.BlockSpec((1, 1, TILE_S, 1), lambda i, j: (j, i, 0, 0)),
            pl.BlockSpec((2, H), lambda i, j: (0, 0)),
            pl.BlockSpec((1, H), lambda i, j: (0, 0)),
            pl.BlockSpec((1, H), lambda i, j: (0, 0)),
        ],
        out_specs=pl.BlockSpec((1, TILE_S, H), lambda i, j: (j, i, 0)),
        out_shape=jax.ShapeDtypeStruct((B, S, H), jnp.float32),
    )(words_embeddings, position_table, tt_t, token_type_table, g, b)
    return out


# speedup vs baseline: 3.9265x; 1.2317x over previous
import jax
import jax.numpy as jnp
from jax.experimental import pallas as pl

B, S, H = 4, 8192, 1024
EPS = 1e-12
TILE_S = 512
NB = S // TILE_S


def _body(w_ref, pos_ref, tt_ref, ttab_ref, g_ref, b_ref, o_ref):
    x = w_ref[0] + pos_ref[...]
    row0 = ttab_ref[0:1, :]
    delta = ttab_ref[1:2, :] - row0
    ttf = tt_ref[0, 0].astype(jnp.float32)
    x = x + row0 + ttf * delta
    u = jnp.mean(x, axis=-1, keepdims=True)
    xc = x - u
    var = jnp.mean(xc * xc, axis=-1, keepdims=True)
    y = xc * jax.lax.rsqrt(var + EPS)
    o_ref[0] = y * g_ref[...] + b_ref[...]


def kernel(words_embeddings, token_type_ids, position_table, token_type_table, ln_weight, ln_bias):
    tt_t = token_type_ids.astype(jnp.int32).reshape(B, NB, TILE_S, 1)
    g = ln_weight.reshape(1, H)
    b = ln_bias.reshape(1, H)
    out = pl.pallas_call(
        _body,
        grid=(NB, B),
        in_specs=[
            pl.BlockSpec((1, TILE_S, H), lambda i, j: (j, i, 0)),
            pl.BlockSpec((TILE_S, H), lambda i, j: (i, 0)),
            pl.BlockSpec((1, 1, TILE_S, 1), lambda i, j: (j, i, 0, 0)),
            pl.BlockSpec((2, H), lambda i, j: (0, 0)),
            pl.BlockSpec((1, H), lambda i, j: (0, 0)),
            pl.BlockSpec((1, H), lambda i, j: (0, 0)),
        ],
        out_specs=pl.BlockSpec((1, TILE_S, H), lambda i, j: (j, i, 0)),
        out_shape=jax.ShapeDtypeStruct((B, S, H), jnp.float32),
    )(words_embeddings, position_table, tt_t, token_type_table, g, b)
    return out


# speedup vs baseline: 4.3950x; 1.1193x over previous
import jax
import jax.numpy as jnp
from jax.experimental import pallas as pl

B, S, H = 4, 8192, 1024
EPS = 1e-12
TILE_S = 1024
NB = S // TILE_S


def _body(w_ref, pos_ref, tt_ref, ttab_ref, g_ref, b_ref, o_ref):
    x = w_ref[0] + pos_ref[...]
    row0 = ttab_ref[0:1, :]
    delta = ttab_ref[1:2, :] - row0
    ttf = tt_ref[0, 0].astype(jnp.float32)
    x = x + row0 + ttf * delta
    u = jnp.mean(x, axis=-1, keepdims=True)
    xc = x - u
    var = jnp.mean(xc * xc, axis=-1, keepdims=True)
    y = xc * jax.lax.rsqrt(var + EPS)
    o_ref[0] = y * g_ref[...] + b_ref[...]


def kernel(words_embeddings, token_type_ids, position_table, token_type_table, ln_weight, ln_bias):
    tt_t = token_type_ids.astype(jnp.int32).reshape(B, NB, TILE_S, 1)
    g = ln_weight.reshape(1, H)
    b = ln_bias.reshape(1, H)
    out = pl.pallas_call(
        _body,
        grid=(NB, B),
        in_specs=[
            pl.BlockSpec((1, TILE_S, H), lambda i, j: (j, i, 0)),
            pl.BlockSpec((TILE_S, H), lambda i, j: (i, 0)),
            pl.BlockSpec((1, 1, TILE_S, 1), lambda i, j: (j, i, 0, 0)),
            pl.BlockSpec((2, H), lambda i, j: (0, 0)),
            pl.BlockSpec((1, H), lambda i, j: (0, 0)),
            pl.BlockSpec((1, H), lambda i, j: (0, 0)),
        ],
        out_specs=pl.BlockSpec((1, TILE_S, H), lambda i, j: (j, i, 0)),
        out_shape=jax.ShapeDtypeStruct((B, S, H), jnp.float32),
    )(words_embeddings, position_table, tt_t, token_type_table, g, b)
    return out


# speedup vs baseline: 5.7526x; 1.3089x over previous
import jax
import jax.numpy as jnp
from jax.experimental import pallas as pl

B, S, H = 4, 8192, 1024
EPS = 1e-12
TILE_S = 2048
NB = S // TILE_S


def _body(w_ref, pos_ref, tt_ref, ttab_ref, g_ref, b_ref, o_ref):
    x = w_ref[0] + pos_ref[...]
    row0 = ttab_ref[0:1, :]
    delta = ttab_ref[1:2, :] - row0
    ttf = tt_ref[0, 0].astype(jnp.float32).reshape(TILE_S, 1)
    x = x + row0 + ttf * delta
    u = jnp.mean(x, axis=-1, keepdims=True)
    xc = x - u
    var = jnp.mean(xc * xc, axis=-1, keepdims=True)
    y = xc * jax.lax.rsqrt(var + EPS)
    o_ref[0] = y * g_ref[...] + b_ref[...]


def kernel(words_embeddings, token_type_ids, position_table, token_type_table, ln_weight, ln_bias):
    tt_t = token_type_ids.astype(jnp.int32).reshape(B, NB, 1, TILE_S)
    g = ln_weight.reshape(1, H)
    b = ln_bias.reshape(1, H)
    out = pl.pallas_call(
        _body,
        grid=(NB, B),
        in_specs=[
            pl.BlockSpec((1, TILE_S, H), lambda i, j: (j, i, 0)),
            pl.BlockSpec((TILE_S, H), lambda i, j: (i, 0)),
            pl.BlockSpec((1, 1, 1, TILE_S), lambda i, j: (j, i, 0, 0)),
            pl.BlockSpec((2, H), lambda i, j: (0, 0)),
            pl.BlockSpec((1, H), lambda i, j: (0, 0)),
            pl.BlockSpec((1, H), lambda i, j: (0, 0)),
        ],
        out_specs=pl.BlockSpec((1, TILE_S, H), lambda i, j: (j, i, 0)),
        out_shape=jax.ShapeDtypeStruct((B, S, H), jnp.float32),
    )(words_embeddings, position_table, tt_t, token_type_table, g, b)
    return out
